# baseline (speedup 1.0000x reference)
import functools

import jax
import jax.numpy as jnp
from jax import lax
from jax.experimental import pallas as pl
from jax.experimental.pallas import tpu as pltpu
from jax.experimental.pallas import tpu_sc as plsc

N = 10000
D = 128
E = 320000
G = 64
C10 = 10

NC = 2
NS = 16
NW = NC * NS
CH = 128
NCH = (E + NW * CH - 1) // (NW * CH)
EPW = NCH * CH
EPAD = NW * EPW
NPAD = 10112
RPT = NPAD // NS
DUMMY = 10104

RB = 400
NRB = N // RB

NCW = 2 * NCH
SLAB = 80


def _sc_mesh():
    return plsc.VectorSubcoreMesh(
        core_axis_name="c", subcore_axis_name="s",
        num_cores=NC, num_subcores=NS)


def _sc_deg(dstp, ones_g, zeros_g):
    @functools.partial(
        pl.kernel,
        out_type=jax.ShapeDtypeStruct((NC, NPAD, D), jnp.float32),
        mesh=_sc_mesh(),
        scratch_types=[
            pltpu.VMEM((NCH, CH), jnp.int32),
            pltpu.VMEM((CH, D), jnp.float32),
            pltpu.VMEM_SHARED((NPAD, D), jnp.float32),
        ],
    )
    def run(dstp_h, ones_h, zeros_h, out_h, dst_v, ones_v, dacc):
        c = lax.axis_index("c")
        s = lax.axis_index("s")
        wid = s * NC + c
        pltpu.sync_copy(dstp_h.at[wid], dst_v)
        pltpu.sync_copy(ones_h, ones_v)
        pltpu.sync_copy(zeros_h, dacc.at[pl.ds(s * RPT, RPT)])
        plsc.subcore_barrier()

        def body(j, carry):
            pltpu.sync_copy(ones_v, dacc.at[dst_v.at[j]], add=True)
            return carry

        lax.fori_loop(0, NCH, body, 0)
        plsc.subcore_barrier()
        pltpu.sync_copy(dacc.at[pl.ds(s * RPT, RPT)],
                        out_h.at[c, pl.ds(s * RPT, RPT)])

    return run(dstp, ones_g, zeros_g)


def _sc_agg(srcp, dstp, xs, zeros_d):
    @functools.partial(
        pl.kernel,
        out_type=jax.ShapeDtypeStruct((NPAD, D), jnp.float32),
        mesh=_sc_mesh(),
        scratch_types=[
            pltpu.VMEM((SLAB, CH), jnp.int32),
            pltpu.VMEM((1, CH), jnp.int32),
            pltpu.VMEM((1, CH), jnp.int32),
            pltpu.VMEM((CH, D), jnp.float32),
            pltpu.VMEM((CH, D), jnp.float32),
            pltpu.VMEM_SHARED((NPAD, D), jnp.float32),
            pltpu.SemaphoreType.DMA,
            pltpu.SemaphoreType.DMA,
            pltpu.SemaphoreType.DMA,
            pltpu.SemaphoreType.DMA,
        ],
    )
    def run(srcp_h, dstp_h, xs_h, zeros_h, out_h,
            src_v, dst_c0, dst_c1, buf0, buf1, acc,
            sem0, sem1, semi0, semi1):
        c = lax.axis_index("c")
        s = lax.axis_index("s")

        @pl.when(c == 0)
        def _core0():
            wbase = s * NCW
            pltpu.sync_copy(zeros_h, acc.at[pl.ds(s * RPT, RPT)])
            pltpu.sync_copy(srcp_h.at[s, pl.ds(0, SLAB)], src_v)
            plsc.subcore_barrier()

            def do_slab(base, nloc):
                pltpu.async_copy(xs_h.at[src_v.at[0]], buf0, sem0)
                pltpu.async_copy(dstp_h.at[wbase + base], dst_c0, semi0)

                def body(t, carry):
                    j = 2 * t
                    g = wbase + base + j
                    pltpu.async_copy(xs_h.at[src_v.at[j + 1]], buf1, sem1)
                    pltpu.async_copy(dstp_h.at[g + 1], dst_c1, semi1)
                    pltpu.make_async_copy(
                        xs_h.at[src_v.at[j]], buf0, sem0).wait()
                    pltpu.make_async_copy(
                        dstp_h.at[g], dst_c0, semi0).wait()
                    pltpu.sync_copy(buf0, acc.at[dst_c0.at[0]], add=True)

                    @pl.when(j + 2 < nloc)
                    def _():
                        pltpu.async_copy(
                            xs_h.at[src_v.at[j + 2]], buf0, sem0)
                        pltpu.async_copy(dstp_h.at[g + 2], dst_c0, semi0)

                    pltpu.make_async_copy(
                        xs_h.at[src_v.at[j + 1]], buf1, sem1).wait()
                    pltpu.make_async_copy(
                        dstp_h.at[g + 1], dst_c1, semi1).wait()
                    pltpu.sync_copy(buf1, acc.at[dst_c1.at[0]], add=True)
                    return carry

                lax.fori_loop(0, nloc // 2, body, 0)

            do_slab(0, SLAB)
            pltpu.sync_copy(srcp_h.at[s, pl.ds(SLAB, NCW - SLAB)],
                            src_v.at[pl.ds(0, NCW - SLAB)])
            do_slab(SLAB, NCW - SLAB)
            plsc.subcore_barrier()
            pltpu.sync_copy(acc.at[pl.ds(s * RPT, RPT)],
                            out_h.at[pl.ds(s * RPT, RPT)])

    return run(srcp, dstp, xs, zeros_d)


def _mm_body(x_ref, w_ref, o_ref):
    o_ref[...] = jnp.dot(x_ref[...], w_ref[...],
                         preferred_element_type=jnp.float32)


def _tc_mm(x, W):
    return pl.pallas_call(
        _mm_body,
        grid=(NRB,),
        in_specs=[
            pl.BlockSpec((RB, D), lambda i: (i, 0)),
            pl.BlockSpec((D, D), lambda i: (0, 0)),
        ],
        out_specs=pl.BlockSpec((RB, D), lambda i: (i, 0)),
        out_shape=jax.ShapeDtypeStruct((N, D), jnp.float32),
    )(x, W)


def _dinv_of(dp):
    return lax.rsqrt(dp[0] + dp[1] + 1.0)[:, 0:1]


def _scale_body(dp_ref, xw_ref, o_ref):
    o_ref[...] = xw_ref[...] * _dinv_of(dp_ref[...])


def _tc_scale(xw, degp):
    return pl.pallas_call(
        _scale_body,
        grid=(NRB,),
        in_specs=[
            pl.BlockSpec((NC, RB, D), lambda i: (0, i, 0)),
            pl.BlockSpec((RB, D), lambda i: (i, 0)),
        ],
        out_specs=pl.BlockSpec((RB, D), lambda i: (i, 0)),
        out_shape=jax.ShapeDtypeStruct((N, D), jnp.float32),
    )(degp, xw)


def _mid_body(dp_ref, xs_ref, p_ref, b_ref, w_ref, o_ref):
    dinv = _dinv_of(dp_ref[...])
    agg = xs_ref[...] + p_ref[...]
    h = jnp.maximum(agg * dinv + b_ref[...], 0.0)
    o_ref[...] = jnp.dot(h, w_ref[...],
                         preferred_element_type=jnp.float32) * dinv


def _tc_mid(xs, p, degp, br, W):
    return pl.pallas_call(
        _mid_body,
        grid=(NRB,),
        in_specs=[
            pl.BlockSpec((NC, RB, D), lambda i: (0, i, 0)),
            pl.BlockSpec((RB, D), lambda i: (i, 0)),
            pl.BlockSpec((RB, D), lambda i: (i, 0)),
            pl.BlockSpec((1, D), lambda i: (0, 0)),
            pl.BlockSpec((D, D), lambda i: (0, 0)),
        ],
        out_specs=pl.BlockSpec((RB, D), lambda i: (i, 0)),
        out_shape=jax.ShapeDtypeStruct((N, D), jnp.float32),
    )(degp, xs, p, br, W)


def _final_body(dp_ref, xs_ref, p_ref, b_ref, batch_ref,
                wf1_ref, bf1_ref, wf2_ref, bf2_ref,
                o_ref, acc_ref, cnt_ref):
    i = pl.program_id(0)

    @pl.when(i == 0)
    def _():
        acc_ref[...] = jnp.zeros_like(acc_ref)
        cnt_ref[...] = jnp.zeros_like(cnt_ref)

    dinv = _dinv_of(dp_ref[...])
    agg = xs_ref[...] + p_ref[...]
    h = jnp.maximum(agg * dinv + b_ref[...], 0.0)
    bidx = batch_ref[0, 0, :]
    onehot = (bidx[:, None] ==
              lax.broadcasted_iota(jnp.int32, (RB, G), 1)).astype(jnp.float32)
    acc_ref[...] += lax.dot_general(
        onehot, h, (((0,), (0,)), ((), ())),
        preferred_element_type=jnp.float32)
    cnt_ref[...] += jnp.sum(onehot, axis=0, keepdims=True)

    @pl.when(i == NRB - 1)
    def _():
        counts = jnp.maximum(cnt_ref[0, :], 1.0)
        hg = acc_ref[...] / counts[:, None]
        hf = jnp.maximum(
            jnp.dot(hg, wf1_ref[...], preferred_element_type=jnp.float32)
            + bf1_ref[...], 0.0)
        o_ref[...] = jnp.dot(hf, wf2_ref[...],
                             preferred_element_type=jnp.float32) + bf2_ref[...]


def _tc_final(xs, p, degp, br, batch3, Wf1, bf1r, Wf2, bf2r):
    return pl.pallas_call(
        _final_body,
        grid=(NRB,),
        in_specs=[
            pl.BlockSpec((NC, RB, D), lambda i: (0, i, 0)),
            pl.BlockSpec((RB, D), lambda i: (i, 0)),
            pl.BlockSpec((RB, D), lambda i: (i, 0)),
            pl.BlockSpec((1, D), lambda i: (0, 0)),
            pl.BlockSpec((1, 1, RB), lambda i: (i, 0, 0)),
            pl.BlockSpec((D, G), lambda i: (0, 0)),
            pl.BlockSpec((1, G), lambda i: (0, 0)),
            pl.BlockSpec((G, C10), lambda i: (0, 0)),
            pl.BlockSpec((1, C10), lambda i: (0, 0)),
        ],
        out_specs=pl.BlockSpec((G, C10), lambda i: (0, 0)),
        out_shape=jax.ShapeDtypeStruct((G, C10), jnp.float32),
        scratch_shapes=[
            pltpu.VMEM((G, D), jnp.float32),
            pltpu.VMEM((1, G), jnp.float32),
        ],
    )(degp, xs, p, br, batch3, Wf1, bf1r, Wf2, bf2r)


def kernel(x, edge_index, batch, W1, b1, W2, b2, W3, b3, Wf1, bf1, Wf2, bf2):
    src = edge_index[0].astype(jnp.int32)
    dst = edge_index[1].astype(jnp.int32)
    npad = EPAD - E
    srcp = jnp.concatenate(
        [src, jnp.zeros((npad,), jnp.int32)]).reshape(NW, NCH, CH)
    dstp = jnp.concatenate(
        [dst, jnp.full((npad,), DUMMY, jnp.int32)]).reshape(NW, NCH, CH)
    srcp2 = srcp.reshape(NS, NCW, CH)
    dstp2 = dstp.reshape(NS * NCW, 1, CH)
    batch3 = batch.astype(jnp.int32).reshape(NRB, 1, RB)
    zeros_d = jnp.zeros((RPT, D), jnp.float32)
    ones_g = jnp.ones((CH, D), jnp.float32)
    b1r = b1.reshape(1, D)
    b2r = b2.reshape(1, D)
    b3r = b3.reshape(1, D)
    bf1r = bf1.reshape(1, G)
    bf2r = bf2.reshape(1, C10)

    degp = _sc_deg(dstp, ones_g, zeros_d)
    xw1 = _tc_mm(x, W1)
    xs1 = _tc_scale(xw1, degp)
    p1 = _sc_agg(srcp2, dstp2, xs1, zeros_d)
    xs2 = _tc_mid(xs1, p1, degp, b1r, W2)
    p2 = _sc_agg(srcp2, dstp2, xs2, zeros_d)
    xs3 = _tc_mid(xs2, p2, degp, b2r, W3)
    p3 = _sc_agg(srcp2, dstp2, xs3, zeros_d)
    return _tc_final(xs3, p3, degp, b3r, batch3, Wf1, bf1r, Wf2, bf2r)


# speedup vs baseline: 1.3700x; 1.3700x over previous
import functools

import jax
import jax.numpy as jnp
from jax import lax
from jax.experimental import pallas as pl
from jax.experimental.pallas import tpu as pltpu
from jax.experimental.pallas import tpu_sc as plsc

N = 10000
D = 128
E = 320000
G = 64
C10 = 10

NC = 2
NS = 16
NW = NC * NS
CH = 128
NCH = (E + NW * CH - 1) // (NW * CH)
EPW = NCH * CH
EPAD = NW * EPW
NPAD = 10112
RPT = NPAD // NS
DUMMY = 10104

RB = 2000
NRB = N // RB

N0 = 108
N1 = 50
NMAX = max(N0, N1)


def _sc_mesh():
    return plsc.VectorSubcoreMesh(
        core_axis_name="c", subcore_axis_name="s",
        num_cores=NC, num_subcores=NS)


def _sc_deg(dstp, ones_g, zeros_g):
    @functools.partial(
        pl.kernel,
        out_type=jax.ShapeDtypeStruct((NC, NPAD, D), jnp.float32),
        mesh=_sc_mesh(),
        scratch_types=[
            pltpu.VMEM((NCH, CH), jnp.int32),
            pltpu.VMEM((CH, D), jnp.float32),
            pltpu.VMEM_SHARED((NPAD, D), jnp.float32),
        ],
    )
    def run(dstp_h, ones_h, zeros_h, out_h, dst_v, ones_v, dacc):
        c = lax.axis_index("c")
        s = lax.axis_index("s")
        wid = s * NC + c
        pltpu.sync_copy(dstp_h.at[wid], dst_v)
        pltpu.sync_copy(ones_h, ones_v)
        pltpu.sync_copy(zeros_h, dacc.at[pl.ds(s * RPT, RPT)])
        plsc.subcore_barrier()

        def body(j, carry):
            pltpu.sync_copy(ones_v, dacc.at[dst_v.at[j]], add=True)
            return carry

        lax.fori_loop(0, NCH, body, 0)
        plsc.subcore_barrier()
        pltpu.sync_copy(dacc.at[pl.ds(s * RPT, RPT)],
                        out_h.at[c, pl.ds(s * RPT, RPT)])

    return run(dstp, ones_g, zeros_g)


def _sc_agg(srcp, dstp, xs, zeros_d):
    @functools.partial(
        pl.kernel,
        out_type=jax.ShapeDtypeStruct((NC, NPAD, D), jnp.float32),
        mesh=_sc_mesh(),
        scratch_types=[
            pltpu.VMEM((NMAX, CH), jnp.int32),
            pltpu.VMEM((1, CH), jnp.int32),
            pltpu.VMEM((1, CH), jnp.int32),
            pltpu.VMEM((CH, D), jnp.float32),
            pltpu.VMEM((CH, D), jnp.float32),
            pltpu.VMEM_SHARED((NPAD, D), jnp.float32),
            pltpu.SemaphoreType.DMA,
            pltpu.SemaphoreType.DMA,
            pltpu.SemaphoreType.DMA,
            pltpu.SemaphoreType.DMA,
        ],
    )
    def run(srcp_h, dstp_h, xs_h, zeros_h, out_h,
            src_v, dst_c0, dst_c1, buf0, buf1, acc,
            sem0, sem1, semi0, semi1):
        c = lax.axis_index("c")
        s = lax.axis_index("s")
        wid = c * NS + s
        wbase = wid * NMAX
        nch = jnp.where(c == 0, N0, N1)
        pltpu.sync_copy(srcp_h.at[wid], src_v)
        pltpu.sync_copy(zeros_h, acc.at[pl.ds(s * RPT, RPT)])
        plsc.subcore_barrier()

        pltpu.async_copy(xs_h.at[src_v.at[0]], buf0, sem0)
        pltpu.async_copy(dstp_h.at[wbase], dst_c0, semi0)

        def body(t, carry):
            j = 2 * t
            pltpu.async_copy(xs_h.at[src_v.at[j + 1]], buf1, sem1)
            pltpu.async_copy(dstp_h.at[wbase + j + 1], dst_c1, semi1)
            pltpu.make_async_copy(xs_h.at[src_v.at[j]], buf0, sem0).wait()
            pltpu.make_async_copy(
                dstp_h.at[wbase + j], dst_c0, semi0).wait()
            pltpu.sync_copy(buf0, acc.at[dst_c0.at[0]], add=True)

            @pl.when(j + 2 < nch)
            def _():
                pltpu.async_copy(xs_h.at[src_v.at[j + 2]], buf0, sem0)
                pltpu.async_copy(dstp_h.at[wbase + j + 2], dst_c0, semi0)

            pltpu.make_async_copy(
                xs_h.at[src_v.at[j + 1]], buf1, sem1).wait()
            pltpu.make_async_copy(
                dstp_h.at[wbase + j + 1], dst_c1, semi1).wait()
            pltpu.sync_copy(buf1, acc.at[dst_c1.at[0]], add=True)
            return carry

        lax.fori_loop(0, nch // 2, body, 0)
        plsc.subcore_barrier()
        pltpu.sync_copy(acc.at[pl.ds(s * RPT, RPT)],
                        out_h.at[c, pl.ds(s * RPT, RPT)])

    return run(srcp, dstp, xs, zeros_d)


def _mm_body(x_ref, w_ref, o_ref):
    o_ref[...] = jnp.dot(x_ref[...], w_ref[...],
                         preferred_element_type=jnp.float32)


def _tc_mm(x, W):
    return pl.pallas_call(
        _mm_body,
        grid=(NRB,),
        in_specs=[
            pl.BlockSpec((RB, D), lambda i: (i, 0)),
            pl.BlockSpec((D, D), lambda i: (0, 0)),
        ],
        out_specs=pl.BlockSpec((RB, D), lambda i: (i, 0)),
        out_shape=jax.ShapeDtypeStruct((N, D), jnp.float32),
    )(x, W)


def _dinv_of(dp):
    return lax.rsqrt(dp[0] + dp[1] + 1.0)[:, 0:1]


def _scale_body(dp_ref, xw_ref, o_ref):
    o_ref[...] = xw_ref[...] * _dinv_of(dp_ref[...])


def _tc_scale(xw, degp):
    return pl.pallas_call(
        _scale_body,
        grid=(NRB,),
        in_specs=[
            pl.BlockSpec((NC, RB, D), lambda i: (0, i, 0)),
            pl.BlockSpec((RB, D), lambda i: (i, 0)),
        ],
        out_specs=pl.BlockSpec((RB, D), lambda i: (i, 0)),
        out_shape=jax.ShapeDtypeStruct((N, D), jnp.float32),
    )(degp, xw)


def _mid_body(dp_ref, xs_ref, p_ref, b_ref, w_ref, o_ref):
    dinv = _dinv_of(dp_ref[...])
    agg = xs_ref[...] + p_ref[0] + p_ref[1]
    h = jnp.maximum(agg * dinv + b_ref[...], 0.0)
    o_ref[...] = jnp.dot(h, w_ref[...],
                         preferred_element_type=jnp.float32) * dinv


def _tc_mid(xs, p, degp, br, W):
    return pl.pallas_call(
        _mid_body,
        grid=(NRB,),
        in_specs=[
            pl.BlockSpec((NC, RB, D), lambda i: (0, i, 0)),
            pl.BlockSpec((RB, D), lambda i: (i, 0)),
            pl.BlockSpec((NC, RB, D), lambda i: (0, i, 0)),
            pl.BlockSpec((1, D), lambda i: (0, 0)),
            pl.BlockSpec((D, D), lambda i: (0, 0)),
        ],
        out_specs=pl.BlockSpec((RB, D), lambda i: (i, 0)),
        out_shape=jax.ShapeDtypeStruct((N, D), jnp.float32),
    )(degp, xs, p, br, W)


def _final_body(dp_ref, xs_ref, p_ref, b_ref, batch_ref,
                wf1_ref, bf1_ref, wf2_ref, bf2_ref,
                o_ref, acc_ref, cnt_ref):
    i = pl.program_id(0)

    @pl.when(i == 0)
    def _():
        acc_ref[...] = jnp.zeros_like(acc_ref)
        cnt_ref[...] = jnp.zeros_like(cnt_ref)

    dinv = _dinv_of(dp_ref[...])
    agg = xs_ref[...] + p_ref[0] + p_ref[1]
    h = jnp.maximum(agg * dinv + b_ref[...], 0.0)
    bidx = batch_ref[0, 0, :]
    onehot = (bidx[:, None] ==
              lax.broadcasted_iota(jnp.int32, (RB, G), 1)).astype(jnp.float32)
    acc_ref[...] += lax.dot_general(
        onehot, h, (((0,), (0,)), ((), ())),
        preferred_element_type=jnp.float32)
    cnt_ref[...] += jnp.sum(onehot, axis=0, keepdims=True)

    @pl.when(i == NRB - 1)
    def _():
        counts = jnp.maximum(cnt_ref[0, :], 1.0)
        hg = acc_ref[...] / counts[:, None]
        hf = jnp.maximum(
            jnp.dot(hg, wf1_ref[...], preferred_element_type=jnp.float32)
            + bf1_ref[...], 0.0)
        o_ref[...] = jnp.dot(hf, wf2_ref[...],
                             preferred_element_type=jnp.float32) + bf2_ref[...]


def _tc_final(xs, p, degp, br, batch3, Wf1, bf1r, Wf2, bf2r):
    return pl.pallas_call(
        _final_body,
        grid=(NRB,),
        in_specs=[
            pl.BlockSpec((NC, RB, D), lambda i: (0, i, 0)),
            pl.BlockSpec((RB, D), lambda i: (i, 0)),
            pl.BlockSpec((NC, RB, D), lambda i: (0, i, 0)),
            pl.BlockSpec((1, D), lambda i: (0, 0)),
            pl.BlockSpec((1, 1, RB), lambda i: (i, 0, 0)),
            pl.BlockSpec((D, G), lambda i: (0, 0)),
            pl.BlockSpec((1, G), lambda i: (0, 0)),
            pl.BlockSpec((G, C10), lambda i: (0, 0)),
            pl.BlockSpec((1, C10), lambda i: (0, 0)),
        ],
        out_specs=pl.BlockSpec((G, C10), lambda i: (0, 0)),
        out_shape=jax.ShapeDtypeStruct((G, C10), jnp.float32),
        scratch_shapes=[
            pltpu.VMEM((G, D), jnp.float32),
            pltpu.VMEM((1, G), jnp.float32),
        ],
    )(degp, xs, p, br, batch3, Wf1, bf1r, Wf2, bf2r)


def kernel(x, edge_index, batch, W1, b1, W2, b2, W3, b3, Wf1, bf1, Wf2, bf2):
    src = edge_index[0].astype(jnp.int32)
    dst = edge_index[1].astype(jnp.int32)
    npad = EPAD - E
    srcp = jnp.concatenate(
        [src, jnp.zeros((npad,), jnp.int32)]).reshape(NW, NCH, CH)
    dstp = jnp.concatenate(
        [dst, jnp.full((npad,), DUMMY, jnp.int32)]).reshape(NW, NCH, CH)
    srcf = srcp.reshape(NW * NCH, CH)
    dstf = dstp.reshape(NW * NCH, CH)

    def _skew(flat, fill):
        a = flat[:NS * N0].reshape(NS, N0, CH)
        b = flat[NS * N0:].reshape(NS, N1, CH)
        a = jnp.pad(a, ((0, 0), (0, NMAX - N0), (0, 0)),
                    constant_values=fill)
        b = jnp.pad(b, ((0, 0), (0, NMAX - N1), (0, 0)),
                    constant_values=fill)
        return jnp.concatenate([a, b], axis=0)

    srcp2 = _skew(srcf, 0)
    dstp2 = _skew(dstf, DUMMY).reshape(NW * NMAX, 1, CH)
    batch3 = batch.astype(jnp.int32).reshape(NRB, 1, RB)
    zeros_d = jnp.zeros((RPT, D), jnp.float32)
    ones_g = jnp.ones((CH, D), jnp.float32)
    b1r = b1.reshape(1, D)
    b2r = b2.reshape(1, D)
    b3r = b3.reshape(1, D)
    bf1r = bf1.reshape(1, G)
    bf2r = bf2.reshape(1, C10)

    degp = _sc_deg(dstp, ones_g, zeros_d)
    xw1 = _tc_mm(x, W1)
    xs1 = _tc_scale(xw1, degp)
    p1 = _sc_agg(srcp2, dstp2, xs1, zeros_d)
    xs2 = _tc_mid(xs1, p1, degp, b1r, W2)
    p2 = _sc_agg(srcp2, dstp2, xs2, zeros_d)
    xs3 = _tc_mid(xs2, p2, degp, b2r, W3)
    p3 = _sc_agg(srcp2, dstp2, xs3, zeros_d)
    return _tc_final(xs3, p3, degp, b3r, batch3, Wf1, bf1r, Wf2, bf2r)
